# Initial kernel scaffold; baseline (speedup 1.0000x reference)
import jax
import jax.numpy as jnp
from jax.experimental import pallas as pl


def kernel(sim_mat, labels):
    raise NotImplementedError("write your pallas kernel here")


# speedup vs baseline: 1.2002x; 1.2002x over previous
import jax
import jax.numpy as jnp
from jax.experimental import pallas as pl

_B = 4096
_ROWS = 256

_THRESH = 0.5
_MARGIN = 0.1
_SCALE_POS = 2.0
_SCALE_NEG = 40.0
_EPS = 1e-5


def _body(sim_ref, labr_ref, labc_ref, out_ref):
    s = sim_ref[...]
    lab_r = labr_ref[...]
    lab_c = labc_ref[:, :1]
    same = lab_c == lab_r

    row_max = jnp.max(s, axis=1, keepdims=True)
    min_pos = jnp.min(jnp.where(same, s, jnp.inf), axis=1, keepdims=True)
    max_neg = jnp.max(jnp.where(same, -jnp.inf, s), axis=1, keepdims=True)
    min_pos = jnp.where(min_pos < row_max - _EPS, min_pos, jnp.inf)

    pos_thr = jnp.minimum(row_max - _EPS, max_neg + _MARGIN)
    neg_thr = min_pos - _MARGIN

    cond = (same & (s < pos_thr)) | (~same & (s > neg_thr))
    scale = jnp.where(same, -_SCALE_POS, _SCALE_NEG)
    bias = jnp.where(same, _THRESH * _SCALE_POS, -_THRESH * _SCALE_NEG)
    e = jnp.where(cond, jnp.exp(s * scale + bias), 0.0)

    psum = jnp.sum(jnp.where(same, e, 0.0), axis=1, keepdims=True)
    nsum = jnp.sum(jnp.where(same, 0.0, e), axis=1, keepdims=True)

    per_row = jnp.log1p(psum) / _SCALE_POS + jnp.log1p(nsum) / _SCALE_NEG
    valid = lab_c != 0
    part = jnp.sum(jnp.where(valid, per_row, 0.0), axis=0, keepdims=True) * (1.0 / _B)

    @pl.when(pl.program_id(0) == 0)
    def _():
        out_ref[...] = jnp.zeros((1, 1), jnp.float32)

    out_ref[...] += part


def kernel(sim_mat, labels):
    lab_r = labels.reshape(1, _B)
    lab_c = jnp.broadcast_to(labels.reshape(_B, 1), (_B, 128))
    out = pl.pallas_call(
        _body,
        grid=(_B // _ROWS,),
        in_specs=[
            pl.BlockSpec((_ROWS, _B), lambda i: (i, 0)),
            pl.BlockSpec((1, _B), lambda i: (0, 0)),
            pl.BlockSpec((_ROWS, 128), lambda i: (i, 0)),
        ],
        out_specs=pl.BlockSpec((1, 1), lambda i: (0, 0)),
        out_shape=jax.ShapeDtypeStruct((1, 1), jnp.float32),
    )(sim_mat, lab_r, lab_c)
    return out[0, 0]


# speedup vs baseline: 1.7760x; 1.4798x over previous
import jax
import jax.numpy as jnp
from jax.experimental import pallas as pl

_B = 4096
_ROWS = 256

_THRESH = 0.5
_MARGIN = 0.1
_SCALE_POS = 2.0
_SCALE_NEG = 40.0
_EPS = 1e-5


def _body(sim_ref, labr_ref, labc_ref, out_ref):
    s = sim_ref[...]
    lab_r = labr_ref[...]
    lab_c = labc_ref[:, :1]
    same = lab_c == lab_r

    row_max = jnp.max(s, axis=1, keepdims=True)
    min_pos = jnp.min(jnp.where(same, s, jnp.inf), axis=1, keepdims=True)
    max_neg = jnp.max(jnp.where(same, -jnp.inf, s), axis=1, keepdims=True)
    min_pos = jnp.where(min_pos < row_max - _EPS, min_pos, jnp.inf)

    pos_thr = jnp.minimum(row_max - _EPS, max_neg + _MARGIN)
    neg_thr = min_pos - _MARGIN

    u_pos_thr = -_SCALE_POS * (pos_thr - _THRESH)
    u_neg_thr = _SCALE_NEG * (neg_thr - _THRESH)
    scale = jnp.where(same, -_SCALE_POS, _SCALE_NEG)
    thr = jnp.where(same, u_pos_thr, u_neg_thr)
    u = (s - _THRESH) * scale
    e = jnp.where(u > thr, jnp.exp(u), 0.0)

    ones = jnp.ones((_B, 1), jnp.float32)
    esum = jax.lax.dot_general(e, ones, (((1,), (0,)), ((), ())),
                               preferred_element_type=jnp.float32)
    ep = jnp.where(same, e, 0.0)
    psum = jax.lax.dot_general(ep, ones, (((1,), (0,)), ((), ())),
                               preferred_element_type=jnp.float32)
    nsum = esum - psum

    per_row = jnp.log1p(psum) / _SCALE_POS + jnp.log1p(nsum) / _SCALE_NEG
    valid = lab_c != 0
    part = jnp.sum(jnp.where(valid, per_row, 0.0), axis=0, keepdims=True) * (1.0 / _B)

    @pl.when(pl.program_id(0) == 0)
    def _():
        out_ref[...] = jnp.zeros((1, 1), jnp.float32)

    out_ref[...] += part


def kernel(sim_mat, labels):
    lab_r = labels.reshape(1, _B)
    lab_c = jnp.broadcast_to(labels.reshape(_B, 1), (_B, 128))
    out = pl.pallas_call(
        _body,
        grid=(_B // _ROWS,),
        in_specs=[
            pl.BlockSpec((_ROWS, _B), lambda i: (i, 0)),
            pl.BlockSpec((1, _B), lambda i: (0, 0)),
            pl.BlockSpec((_ROWS, 128), lambda i: (i, 0)),
        ],
        out_specs=pl.BlockSpec((1, 1), lambda i: (0, 0)),
        out_shape=jax.ShapeDtypeStruct((1, 1), jnp.float32),
    )(sim_mat, lab_r, lab_c)
    return out[0, 0]


# speedup vs baseline: 1.9100x; 1.0755x over previous
import jax
import jax.numpy as jnp
from jax.experimental import pallas as pl

_B = 4096
_NUM_CLASSES = 64
_ROWS = 256

_THRESH = 0.5
_MARGIN = 0.1
_SCALE_POS = 2.0
_SCALE_NEG = 40.0
_EPS = 1e-5
_LOG2E = 1.4426950408889634
_THR_CAP = 88.0


def _body(sim_ref, labr_ref, labc_ref, c_ref, out_ref):
    s = sim_ref[...]
    lab_r = labr_ref[...]
    lab_c = labc_ref[:, :1]
    same = lab_c == lab_r

    row_max = jnp.max(s, axis=1, keepdims=True)
    min_pos = jnp.min(jnp.where(same, s, jnp.inf), axis=1, keepdims=True)
    max_neg = jnp.max(jnp.where(same, -jnp.inf, s), axis=1, keepdims=True)
    min_pos = jnp.where(min_pos < row_max - _EPS, min_pos, jnp.inf)

    pos_thr = jnp.minimum(row_max - _EPS, max_neg + _MARGIN)
    neg_thr = min_pos - _MARGIN

    u_pos_thr = jnp.minimum(-_SCALE_POS * (pos_thr - _THRESH), _THR_CAP)
    u_neg_thr = jnp.minimum(_SCALE_NEG * (neg_thr - _THRESH), _THR_CAP)

    a_pos = -_SCALE_POS * _LOG2E
    a_neg = _SCALE_NEG * _LOG2E
    b_pos = (_THRESH * _SCALE_POS - u_pos_thr) * _LOG2E
    b_neg = (-_THRESH * _SCALE_NEG - u_neg_thr) * _LOG2E
    a = jnp.where(same, a_pos, a_neg)
    b = jnp.where(same, b_pos, b_neg)
    w = s * a + b
    e = jnp.where(w > 0.0, jnp.exp2(w), 0.0)

    g = jax.lax.dot_general(e, c_ref[...], (((1,), (0,)), ((), ())),
                            preferred_element_type=jnp.float32)
    esum = jnp.sum(g, axis=1, keepdims=True)
    rowhot = lab_c == jax.lax.broadcasted_iota(jnp.int32, (1, _NUM_CLASSES), 1)
    psum_raw = jnp.sum(jnp.where(rowhot, g, 0.0), axis=1, keepdims=True)

    psum = psum_raw * jnp.exp(u_pos_thr)
    nsum = (esum - psum_raw) * jnp.exp(u_neg_thr)

    per_row = jnp.log1p(psum) / _SCALE_POS + jnp.log1p(nsum) / _SCALE_NEG
    valid = lab_c != 0
    part = jnp.sum(jnp.where(valid, per_row, 0.0), axis=0, keepdims=True) * (1.0 / _B)

    @pl.when(pl.program_id(0) == 0)
    def _():
        out_ref[...] = jnp.zeros((1, 1), jnp.float32)

    out_ref[...] += part


def kernel(sim_mat, labels):
    lab_r = labels.reshape(1, _B)
    lab_c = jnp.broadcast_to(labels.reshape(_B, 1), (_B, 128))
    c_mat = (labels.reshape(_B, 1)
             == jnp.arange(_NUM_CLASSES, dtype=jnp.int32).reshape(1, _NUM_CLASSES)
             ).astype(jnp.float32)
    out = pl.pallas_call(
        _body,
        grid=(_B // _ROWS,),
        in_specs=[
            pl.BlockSpec((_ROWS, _B), lambda i: (i, 0)),
            pl.BlockSpec((1, _B), lambda i: (0, 0)),
            pl.BlockSpec((_ROWS, 128), lambda i: (i, 0)),
            pl.BlockSpec((_B, _NUM_CLASSES), lambda i: (0, 0)),
        ],
        out_specs=pl.BlockSpec((1, 1), lambda i: (0, 0)),
        out_shape=jax.ShapeDtypeStruct((1, 1), jnp.float32),
    )(sim_mat, lab_r, lab_c, c_mat)
    return out[0, 0]
